# baseline (speedup 1.0000x reference)
import functools

import jax
import jax.numpy as jnp
import numpy as np
from jax import lax
from jax.experimental import pallas as pl
from jax.experimental.pallas import tpu as pltpu
from jax.experimental.pallas import tpu_sc as plsc

_MAP_SIZES = [64, 32, 16, 8, 4, 2]
_NB = 6
_B = 16
_NOBJ = 64
_NCLS = 21
_CP = 32
_LP = 8
_SEG_H = 256
_TOTAL = sum(_B * _NB * ms * ms for ms in _MAP_SIZES)
_CLS_RB = 5760
_CLS_STEPS = _TOTAL // _CLS_RB
_SEG_BH = 64
_SEG_STEPS = _B * (_SEG_H // _SEG_BH)

_LAYER_OFF = [0, 393216, 491520, 516096]
_LAYER_BSTRIDE = [24576, 6144, 1536, 384]

_CLS_ROWS = _TOTAL * _CP // 128
_LOC_ROWS = _TOTAL * _LP // 128
_CLS_RB2 = _CLS_ROWS // _CLS_STEPS

_M8 = np.zeros((128, 8), np.float32)
for _a in range(4):
    _M8[32 * _a:32 * _a + 32, _a] = 1.0
    _M8[32 * _a, 4 + _a] = 1.0


def _dense_body(x_ref, m_ref, seg_ref, lab_ref, acc_ref):
    i = pl.program_id(0)

    @pl.when(i == 0)
    def _():
        acc_ref[0, 0] = 0.0
        acc_ref[0, 1] = 0.0

    x = x_ref[...]
    e = jnp.exp(x).astype(jnp.bfloat16)
    y = jnp.dot(e, m_ref[...], preferred_element_type=jnp.float32)
    ly = jnp.log(y)
    acc_ref[0, 0] += jnp.sum(ly[:, :4]) - jnp.sum(ly[:, 4:])

    @pl.when(i < _SEG_STEPS)
    def _():
        lab = lab_ref[0]
        x0 = seg_ref[0, 0]
        se = jnp.exp(x0)
        xl = jnp.where(lab == 0, x0, 0.0)
        for c in range(1, _NCLS):
            xc = seg_ref[0, c]
            se = se + jnp.exp(xc)
            xl = jnp.where(lab == c, xc, xl)
        acc_ref[0, 1] += jnp.sum(jnp.log(se)) - jnp.sum(xl)


def _take16(x, idx):
    dnums = lax.GatherDimensionNumbers(
        offset_dims=(), collapsed_slice_dims=(0,), start_index_map=(0,))
    return lax.gather(x, idx[:, None], dnums, slice_sizes=(1,),
                      mode=lax.GatherScatterMode.PROMISE_IN_BOUNDS)


def _sc_body(cls1d, loc1d, idxt, clsb, gtt, dft, out,
             liv, piv, biv, cbv, gtv, dfv, g0v, glv, lgv, outv, sem):
    w = lax.axis_index("s") * 2 + lax.axis_index("c")

    @pl.when(w < _B)
    def _():
        b = w
        pltpu.sync_copy(idxt.at[0, b], liv)
        pltpu.sync_copy(idxt.at[1, b], piv)
        pltpu.sync_copy(idxt.at[2, b], biv)
        pltpu.sync_copy(clsb.at[b], cbv)
        for c in range(4):
            pltpu.sync_copy(gtt.at[c, b], gtv.at[c])
            pltpu.sync_copy(dft.at[c, b], dfv.at[c])

        iota = lax.iota(jnp.int32, 16)
        flats = []
        labs = []
        handles = []
        for v in range(4):
            ly = liv[pl.ds(16 * v, 16)]
            ps = piv[pl.ds(16 * v, 16)]
            bx = biv[pl.ds(16 * v, 16)]
            lb = cbv[pl.ds(16 * v, 16)]
            off = jnp.where(
                ly == 0, _LAYER_OFF[0],
                jnp.where(ly == 1, _LAYER_OFF[1],
                          jnp.where(ly == 2, _LAYER_OFF[2], _LAYER_OFF[3])))
            bst = jnp.where(
                ly == 0, _LAYER_BSTRIDE[0],
                jnp.where(ly == 1, _LAYER_BSTRIDE[1],
                          jnp.where(ly == 2, _LAYER_BSTRIDE[2],
                                    _LAYER_BSTRIDE[3])))
            f = off + b * bst + ps * _NB + bx
            flats.append(f)
            labs.append(lb)
            handles.append(
                pltpu.async_copy(cls1d.at[f * _CP], g0v.at[v], sem))
            handles.append(
                pltpu.async_copy(cls1d.at[f * _CP + lb], glv.at[v], sem))
            for c in range(4):
                handles.append(
                    pltpu.async_copy(loc1d.at[f * _LP + c], lgv.at[v, c], sem))

        wins = []
        for v in range(4):
            dup = jnp.zeros((16,), jnp.bool_)
            for k in range(1, 16):
                rolled = _take16(flats[v], (iota + k) & 15)
                dup = dup | ((rolled == flats[v]) & (iota < 16 - k))
            for u in range(v + 1, 4):
                for k in range(16):
                    rolled = _take16(flats[u], (iota + k) & 15)
                    dup = dup | (rolled == flats[v])
            wins.append(jnp.logical_not(dup))

        for h in handles:
            h.wait()

        cls_corr = jnp.float32(0.0)
        loc_sum = jnp.float32(0.0)
        npos = jnp.float32(0.0)
        for v in range(4):
            winf = wins[v].astype(jnp.float32)
            posf = (wins[v] & (labs[v] > 0)).astype(jnp.float32)
            cls_corr = cls_corr + jnp.sum((g0v[v] - glv[v]) * winf)
            sl1 = jnp.zeros((16,), jnp.float32)
            for c in range(4):
                gtc = gtv[c, pl.ds(16 * v, 16)]
                dfc = dfv[c, pl.ds(16 * v, 16)]
                lv = (gtc - dfc) / jnp.float32(0.1)
                d = jnp.abs(lgv[v, c] - lv)
                sl1 = sl1 + jnp.where(d < 1.0, 0.5 * d * d, d - 0.5)
            loc_sum = loc_sum + jnp.sum(sl1 * posf)
            npos = npos + jnp.sum(posf)

        outv[...] = jnp.where(
            iota == 0, cls_corr,
            jnp.where(iota == 1, loc_sum,
                      jnp.where(iota == 2, npos, jnp.float32(0.0))))
        pltpu.sync_copy(outv, out.at[b])


def kernel(Loc, Cls, Seg, gt_box_batch, df_box_batch, idx_batch, cls_batch,
           bat_s, mining, seg_label):
    clsp = jnp.pad(Cls, ((0, 0), (0, _CP - _NCLS)),
                   constant_values=-1e30).reshape(_CLS_ROWS, 128)
    locp = jnp.pad(Loc, ((0, 0), (0, _LP - 4))).reshape(_LOC_ROWS, 128)

    def _seg_i(i):
        j = jnp.minimum(i, _SEG_STEPS - 1)
        return j // (_SEG_H // _SEG_BH), j % (_SEG_H // _SEG_BH)

    def _seg_map(i):
        bi, hi = _seg_i(i)
        return (bi, 0, hi, 0)

    def _lab_map(i):
        bi, hi = _seg_i(i)
        return (bi, hi, 0)

    acc = pl.pallas_call(
        _dense_body,
        grid=(_CLS_STEPS,),
        in_specs=[
            pl.BlockSpec((_CLS_RB2, 128), lambda i: (i, 0)),
            pl.BlockSpec((128, 8), lambda i: (0, 0)),
            pl.BlockSpec((1, _NCLS, _SEG_BH, _SEG_H), _seg_map),
            pl.BlockSpec((1, _SEG_BH, _SEG_H), _lab_map),
        ],
        out_specs=pl.BlockSpec((1, 2), lambda i: (0, 0),
                               memory_space=pltpu.SMEM),
        out_shape=jax.ShapeDtypeStruct((1, 2), jnp.float32),
    )(clsp, jnp.asarray(_M8, dtype=jnp.bfloat16), Seg,
      seg_label.astype(jnp.int32))
    cls_dense = acc[0, 0]
    seg_sum = acc[0, 1]

    idxt = jnp.transpose(idx_batch[..., 1:].astype(jnp.int32), (2, 0, 1))
    gtt = jnp.transpose(gt_box_batch, (2, 0, 1))
    dft = jnp.transpose(df_box_batch, (2, 0, 1))
    mesh = plsc.VectorSubcoreMesh(core_axis_name="c", subcore_axis_name="s")
    parts = pl.kernel(
        _sc_body,
        mesh=mesh,
        compiler_params=pltpu.CompilerParams(needs_layout_passes=False),
        out_type=jax.ShapeDtypeStruct((_B, 16), jnp.float32),
        scratch_types=[
            pltpu.VMEM((_NOBJ,), jnp.int32),
            pltpu.VMEM((_NOBJ,), jnp.int32),
            pltpu.VMEM((_NOBJ,), jnp.int32),
            pltpu.VMEM((_NOBJ,), jnp.int32),
            pltpu.VMEM((4, _NOBJ), jnp.float32),
            pltpu.VMEM((4, _NOBJ), jnp.float32),
            pltpu.VMEM((4, 16), jnp.float32),
            pltpu.VMEM((4, 16), jnp.float32),
            pltpu.VMEM((4, 4, 16), jnp.float32),
            pltpu.VMEM((16,), jnp.float32),
            pltpu.SemaphoreType.DMA,
        ],
    )(clsp.reshape(-1), locp.reshape(-1), idxt,
      cls_batch.astype(jnp.int32), gtt, dft)

    cls_corr = jnp.sum(parts[:, 0])
    loc_sum = jnp.sum(parts[:, 1])
    npos = jnp.sum(parts[:, 2])

    cls_loss = (cls_dense + cls_corr) / jnp.float32(_TOTAL)
    loc_loss = loc_sum / jnp.maximum(npos, 1.0)
    seg_loss = seg_sum / jnp.float32(_B * _SEG_H * _SEG_H)
    return cls_loss + loc_loss + seg_loss


# speedup vs baseline: 1.0931x; 1.0931x over previous
import functools

import jax
import jax.numpy as jnp
import numpy as np
from jax import lax
from jax.experimental import pallas as pl
from jax.experimental.pallas import tpu as pltpu
from jax.experimental.pallas import tpu_sc as plsc

_MAP_SIZES = [64, 32, 16, 8, 4, 2]
_NB = 6
_B = 16
_NOBJ = 64
_NCLS = 21
_CP = 32
_LP = 8
_SEG_H = 256
_TOTAL = sum(_B * _NB * ms * ms for ms in _MAP_SIZES)
_CLS_ROWS = _TOTAL * _CP // 128
_LOC_ROWS = _TOTAL * _LP // 128
_CLS_STEPS = 91
_CLS_RB2 = _CLS_ROWS // _CLS_STEPS
_SEG_BH = 64
_SEG_STEPS = _B * (_SEG_H // _SEG_BH)
_K = 12
_STEPS = 96
_NPAD = _STEPS * _K

_LAYER_OFF = [0, 393216, 491520, 516096]
_LAYER_BSTRIDE = [24576, 6144, 1536, 384]

_M8 = np.zeros((128, 8), np.float32)
for _a in range(4):
    _M8[32 * _a:32 * _a + 32, _a] = 1.0
    _M8[32 * _a, 4 + _a] = 1.0


def _fused_body(gc_ref, gl_ref, x_ref, m_ref, seg_ref, lab_ref, *refs):
    cbs = refs[:_K]
    lbs = refs[_K:2 * _K]
    (subc_r, labl_r, lanebc_r, subl_r, lanebl_r,
     win_r, pos_r, t0_r, t1_r, t2_r, t3_r) = refs[2 * _K:-1]
    acc_ref = refs[-1]
    i = pl.program_id(0)

    @pl.when(i == 0)
    def _():
        for c in range(8):
            acc_ref[0, c] = 0.0

    @pl.when(i < _CLS_STEPS)
    def _():
        x = x_ref[...]
        e = jnp.exp(x).astype(jnp.bfloat16)
        y = jnp.dot(e, m_ref[...], preferred_element_type=jnp.float32)
        ly = jnp.log(y)
        acc_ref[0, 0] += jnp.sum(ly[:, :4]) - jnp.sum(ly[:, 4:])

    @pl.when(i < _SEG_STEPS)
    def _():
        lab = lab_ref[0]
        x0 = seg_ref[0, 0]
        se = jnp.exp(x0)
        xl = jnp.where(lab == 0, x0, 0.0)
        for c in range(1, _NCLS):
            xc = seg_ref[0, c]
            se = se + jnp.exp(xc)
            xl = jnp.where(lab == c, xc, xl)
        acc_ref[0, 1] += jnp.sum(jnp.log(se)) - jnp.sum(xl)

    iota = lax.broadcasted_iota(jnp.int32, (1, 128), 1)
    a_cls = jnp.float32(0.0)
    a_loc = jnp.float32(0.0)
    a_n = jnp.float32(0.0)
    for j in range(_K):
        idx = i * _K + j
        subc = subc_r[idx]
        labl = labl_r[idx]
        lanebc = lanebc_r[idx]
        subl = subl_r[idx]
        lanebl = lanebl_r[idx]
        w = win_r[idx]
        p = pos_r[idx]
        x = cbs[j][pl.ds(subc, 1), :]
        x0 = jnp.sum(jnp.where(iota == lanebc, x, 0.0))
        xl = jnp.sum(jnp.where(iota == labl, x, 0.0))
        a_cls = a_cls + w * (x0 - xl)
        l = lbs[j][pl.ds(subl, 1), :]
        t = jnp.where(iota == lanebl, t0_r[idx],
                      jnp.where(iota == lanebl + 1, t1_r[idx],
                                jnp.where(iota == lanebl + 2, t2_r[idx],
                                          t3_r[idx])))
        msk = (iota >= lanebl) & (iota < lanebl + 4)
        d = jnp.abs(l - t)
        hub = jnp.where(msk, jnp.where(d < 1.0, 0.5 * d * d, d - 0.5), 0.0)
        a_loc = a_loc + p * jnp.sum(hub)
        a_n = a_n + p
    acc_ref[0, 2] += a_cls
    acc_ref[0, 3] += a_loc
    acc_ref[0, 4] += a_n


def _take16(x, idx):
    dnums = lax.GatherDimensionNumbers(
        offset_dims=(), collapsed_slice_dims=(0,), start_index_map=(0,))
    return lax.gather(x, idx[:, None], dnums, slice_sizes=(1,),
                      mode=lax.GatherScatterMode.PROMISE_IN_BOUNDS)


def _sc_body(idxt, clsb, gtt, dft,
             o_gc, o_gl, o_subc, o_labl, o_lanebc, o_subl, o_lanebl,
             o_win, o_pos, o_t0, o_t1, o_t2, o_t3,
             liv, piv, biv, cbv, gtv, dfv,
             s_gc, s_gl, s_subc, s_labl, s_lanebc, s_subl, s_lanebl,
             s_win, s_pos, s_t0, s_t1, s_t2, s_t3, zi, zf):
    w = lax.axis_index("s") * 2 + lax.axis_index("c")

    @pl.when(w < _B)
    def _():
        b = w
        pltpu.sync_copy(idxt.at[0, b], liv)
        pltpu.sync_copy(idxt.at[1, b], piv)
        pltpu.sync_copy(idxt.at[2, b], biv)
        pltpu.sync_copy(clsb.at[b], cbv)
        for c in range(4):
            pltpu.sync_copy(gtt.at[c, b], gtv.at[c])
            pltpu.sync_copy(dft.at[c, b], dfv.at[c])

        iota = lax.iota(jnp.int32, 16)
        flats = []
        labs = []
        for v in range(4):
            ly = liv[pl.ds(16 * v, 16)]
            ps = piv[pl.ds(16 * v, 16)]
            bx = biv[pl.ds(16 * v, 16)]
            lb = cbv[pl.ds(16 * v, 16)]
            off = jnp.where(
                ly == 0, _LAYER_OFF[0],
                jnp.where(ly == 1, _LAYER_OFF[1],
                          jnp.where(ly == 2, _LAYER_OFF[2], _LAYER_OFF[3])))
            bst = jnp.where(
                ly == 0, _LAYER_BSTRIDE[0],
                jnp.where(ly == 1, _LAYER_BSTRIDE[1],
                          jnp.where(ly == 2, _LAYER_BSTRIDE[2],
                                    _LAYER_BSTRIDE[3])))
            flats.append(off + b * bst + ps * _NB + bx)
            labs.append(lb)

        for v in range(4):
            dup = jnp.zeros((16,), jnp.bool_)
            for k in range(1, 16):
                rolled = _take16(flats[v], (iota + k) & 15)
                dup = dup | ((rolled == flats[v]) & (iota < 16 - k))
            for u in range(v + 1, 4):
                for k in range(16):
                    rolled = _take16(flats[u], (iota + k) & 15)
                    dup = dup | (rolled == flats[v])
            win = jnp.logical_not(dup)
            f = flats[v]
            lanebc = (f & 3) * _CP
            sl = pl.ds(16 * v, 16)
            s_gc[sl] = f >> 5
            s_gl[sl] = f >> 7
            s_subc[sl] = (f >> 2) & 7
            s_labl[sl] = lanebc + labs[v]
            s_lanebc[sl] = lanebc
            s_subl[sl] = (f >> 4) & 7
            s_lanebl[sl] = (f & 15) * _LP
            s_win[sl] = win.astype(jnp.float32)
            s_pos[sl] = (win & (labs[v] > 0)).astype(jnp.float32)
            for c, stc in enumerate((s_t0, s_t1, s_t2, s_t3)):
                gtc = gtv[c, sl]
                dfc = dfv[c, sl]
                stc[sl] = (gtc - dfc) / jnp.float32(0.1)

        base = w * _NOBJ
        outs = (o_gc, o_gl, o_subc, o_labl, o_lanebc, o_subl, o_lanebl,
                o_win, o_pos, o_t0, o_t1, o_t2, o_t3)
        scr = (s_gc, s_gl, s_subc, s_labl, s_lanebc, s_subl, s_lanebl,
               s_win, s_pos, s_t0, s_t1, s_t2, s_t3)
        for o, s in zip(outs, scr):
            pltpu.sync_copy(s, o.at[pl.ds(base, _NOBJ)])

        @pl.when(w == 0)
        def _():
            for t in range(8):
                zi[pl.ds(16 * t, 16)] = jnp.zeros((16,), jnp.int32)
                zf[pl.ds(16 * t, 16)] = jnp.zeros((16,), jnp.float32)
            for o in (o_gc, o_gl, o_subc, o_labl, o_lanebc, o_subl,
                      o_lanebl):
                pltpu.sync_copy(zi, o.at[pl.ds(_B * _NOBJ, 128)])
            for o in (o_win, o_pos, o_t0, o_t1, o_t2, o_t3):
                pltpu.sync_copy(zf, o.at[pl.ds(_B * _NOBJ, 128)])


def kernel(Loc, Cls, Seg, gt_box_batch, df_box_batch, idx_batch, cls_batch,
           bat_s, mining, seg_label):
    clsp = jnp.pad(Cls, ((0, 0), (0, _CP - _NCLS)),
                   constant_values=-1e30).reshape(_CLS_ROWS, 128)
    locp = jnp.pad(Loc, ((0, 0), (0, _LP - 4))).reshape(_LOC_ROWS, 128)

    idxt = jnp.transpose(idx_batch[..., 1:].astype(jnp.int32), (2, 0, 1))
    gtt = jnp.transpose(gt_box_batch, (2, 0, 1))
    dft = jnp.transpose(df_box_batch, (2, 0, 1))
    mesh = plsc.VectorSubcoreMesh(core_axis_name="c", subcore_axis_name="s")
    i32v = jax.ShapeDtypeStruct((_NPAD,), jnp.int32)
    f32v = jax.ShapeDtypeStruct((_NPAD,), jnp.float32)
    (gc, gl, subc, labl, lanebc, subl, lanebl,
     win, pos, t0, t1, t2, t3) = pl.kernel(
        _sc_body,
        mesh=mesh,
        compiler_params=pltpu.CompilerParams(needs_layout_passes=False),
        out_type=(i32v, i32v, i32v, i32v, i32v, i32v, i32v,
                  f32v, f32v, f32v, f32v, f32v, f32v),
        scratch_types=(
            [pltpu.VMEM((_NOBJ,), jnp.int32)] * 4
            + [pltpu.VMEM((4, _NOBJ), jnp.float32)] * 2
            + [pltpu.VMEM((_NOBJ,), jnp.int32)] * 7
            + [pltpu.VMEM((_NOBJ,), jnp.float32)] * 6
            + [pltpu.VMEM((128,), jnp.int32),
               pltpu.VMEM((128,), jnp.float32)]
        ),
    )(idxt, cls_batch.astype(jnp.int32), gtt, dft)

    def _seg_i(i):
        j = jnp.minimum(i, _SEG_STEPS - 1)
        return j // (_SEG_H // _SEG_BH), j % (_SEG_H // _SEG_BH)

    def _seg_map(i, gc_ref, gl_ref):
        bi, hi = _seg_i(i)
        return (bi, 0, hi, 0)

    def _lab_map(i, gc_ref, gl_ref):
        bi, hi = _seg_i(i)
        return (bi, hi, 0)

    acc = pl.pallas_call(
        _fused_body,
        grid_spec=pltpu.PrefetchScalarGridSpec(
            num_scalar_prefetch=2,
            grid=(_STEPS,),
            in_specs=[
                pl.BlockSpec((_CLS_RB2, 128),
                             lambda i, gc_ref, gl_ref:
                             (jnp.minimum(i, _CLS_STEPS - 1), 0)),
                pl.BlockSpec((128, 8), lambda i, gc_ref, gl_ref: (0, 0)),
                pl.BlockSpec((1, _NCLS, _SEG_BH, _SEG_H), _seg_map),
                pl.BlockSpec((1, _SEG_BH, _SEG_H), _lab_map),
            ] + [
                pl.BlockSpec(
                    (8, 128),
                    functools.partial(
                        lambda i, gc_ref, gl_ref, j:
                        (gc_ref[i * _K + j], 0), j=j))
                for j in range(_K)
            ] + [
                pl.BlockSpec(
                    (8, 128),
                    functools.partial(
                        lambda i, gc_ref, gl_ref, j:
                        (gl_ref[i * _K + j], 0), j=j))
                for j in range(_K)
            ] + [pl.BlockSpec(memory_space=pltpu.SMEM)] * 11,
            out_specs=pl.BlockSpec((1, 8), lambda i, gc_ref, gl_ref: (0, 0),
                                   memory_space=pltpu.SMEM),
        ),
        out_shape=jax.ShapeDtypeStruct((1, 8), jnp.float32),
    )(gc, gl, clsp, jnp.asarray(_M8, dtype=jnp.bfloat16), Seg,
      seg_label.astype(jnp.int32), *([clsp] * _K), *([locp] * _K),
      subc, labl, lanebc, subl, lanebl, win, pos, t0, t1, t2, t3)

    cls_loss = (acc[0, 0] + acc[0, 2]) / jnp.float32(_TOTAL)
    loc_loss = acc[0, 3] / jnp.maximum(acc[0, 4], 1.0)
    seg_loss = acc[0, 1] / jnp.float32(_B * _SEG_H * _SEG_H)
    return cls_loss + loc_loss + seg_loss


# speedup vs baseline: 2.6066x; 2.3846x over previous
import functools

import jax
import jax.numpy as jnp
import numpy as np
from jax import lax
from jax.experimental import pallas as pl
from jax.experimental.pallas import tpu as pltpu
from jax.experimental.pallas import tpu_sc as plsc

_MAP_SIZES = [64, 32, 16, 8, 4, 2]
_NB = 6
_B = 16
_NOBJ = 64
_NCLS = 21
_SEG_H = 256
_TOTAL = sum(_B * _NB * ms * ms for ms in _MAP_SIZES)
_NSPLIT = 3
_CLS_STEPS = 91
_CLS_RB = _TOTAL // (_NSPLIT * _CLS_STEPS)
_SEG_BH = 64
_SEG_STEPS = _B * (_SEG_H // _SEG_BH)
_K = 12
_STEPS = 96
_NPAD = _STEPS * _K

_LAYER_OFF = [0, 393216, 491520, 516096]
_LAYER_BSTRIDE = [24576, 6144, 1536, 384]


def _fused_body(g_ref, *refs):
    xs = refs[:_NSPLIT]
    seg_ref, lab_ref = refs[_NSPLIT:_NSPLIT + 2]
    cbs = refs[_NSPLIT + 2:_NSPLIT + 2 + _K]
    lbs = refs[_NSPLIT + 2 + _K:_NSPLIT + 2 + 2 * _K]
    (sub_r, lab_r, win_r, pos_r, t0_r, t1_r, t2_r, t3_r) = \
        refs[_NSPLIT + 2 + 2 * _K:-1]
    acc_ref = refs[-1]
    i = pl.program_id(0)

    @pl.when(i == 0)
    def _():
        for c in range(8):
            acc_ref[0, c] = 0.0

    @pl.when(i < _CLS_STEPS)
    def _():
        total = jnp.float32(0.0)
        for x_ref in xs:
            x = x_ref[...]
            s = jnp.sum(jnp.exp(x), axis=1)
            total = total + jnp.sum(jnp.log(s)) - jnp.sum(x[:, 0])
        acc_ref[0, 0] += total

    @pl.when(i < _SEG_STEPS)
    def _():
        lab = lab_ref[0]
        x0 = seg_ref[0, 0]
        se = jnp.exp(x0)
        xl = jnp.where(lab == 0, x0, 0.0)
        for c in range(1, _NCLS):
            xc = seg_ref[0, c]
            se = se + jnp.exp(xc)
            xl = jnp.where(lab == c, xc, xl)
        acc_ref[0, 1] += jnp.sum(jnp.log(se)) - jnp.sum(xl)

    lane = lax.broadcasted_iota(jnp.int32, (1, _NCLS), 1)
    lane4 = lax.broadcasted_iota(jnp.int32, (1, 4), 1)
    a_cls = jnp.float32(0.0)
    a_loc = jnp.float32(0.0)
    a_n = jnp.float32(0.0)
    for j in range(_K):
        idx = i * _K + j
        sub = sub_r[idx]
        lab = lab_r[idx]
        w = win_r[idx]
        p = pos_r[idx]
        x = cbs[j][pl.ds(sub, 1), :]
        x0 = jnp.sum(jnp.where(lane == 0, x, 0.0))
        xl = jnp.sum(jnp.where(lane == lab, x, 0.0))
        a_cls = a_cls + w * (x0 - xl)
        l = lbs[j][pl.ds(sub, 1), :]
        t = jnp.where(lane4 == 0, t0_r[idx],
                      jnp.where(lane4 == 1, t1_r[idx],
                                jnp.where(lane4 == 2, t2_r[idx], t3_r[idx])))
        d = jnp.abs(l - t)
        a_loc = a_loc + p * jnp.sum(jnp.where(d < 1.0, 0.5 * d * d, d - 0.5))
        a_n = a_n + p
    acc_ref[0, 2] += a_cls
    acc_ref[0, 3] += a_loc
    acc_ref[0, 4] += a_n


def _take16(x, idx):
    dnums = lax.GatherDimensionNumbers(
        offset_dims=(), collapsed_slice_dims=(0,), start_index_map=(0,))
    return lax.gather(x, idx[:, None], dnums, slice_sizes=(1,),
                      mode=lax.GatherScatterMode.PROMISE_IN_BOUNDS)


def _sc_body(idxt, clsb, gtt, dft,
             o_g, o_sub, o_lab, o_win, o_pos, o_t0, o_t1, o_t2, o_t3,
             liv, piv, biv, cbv, gtv, dfv,
             s_g, s_sub, s_lab, s_win, s_pos, s_t0, s_t1, s_t2, s_t3,
             zi, zf):
    w = lax.axis_index("s") * 2 + lax.axis_index("c")

    @pl.when(w < _B)
    def _():
        b = w
        pltpu.sync_copy(idxt.at[0, b], liv)
        pltpu.sync_copy(idxt.at[1, b], piv)
        pltpu.sync_copy(idxt.at[2, b], biv)
        pltpu.sync_copy(clsb.at[b], cbv)
        for c in range(4):
            pltpu.sync_copy(gtt.at[c, b], gtv.at[c])
            pltpu.sync_copy(dft.at[c, b], dfv.at[c])

        iota = lax.iota(jnp.int32, 16)
        flats = []
        labs = []
        for v in range(4):
            ly = liv[pl.ds(16 * v, 16)]
            ps = piv[pl.ds(16 * v, 16)]
            bx = biv[pl.ds(16 * v, 16)]
            lb = cbv[pl.ds(16 * v, 16)]
            off = jnp.where(
                ly == 0, _LAYER_OFF[0],
                jnp.where(ly == 1, _LAYER_OFF[1],
                          jnp.where(ly == 2, _LAYER_OFF[2], _LAYER_OFF[3])))
            bst = jnp.where(
                ly == 0, _LAYER_BSTRIDE[0],
                jnp.where(ly == 1, _LAYER_BSTRIDE[1],
                          jnp.where(ly == 2, _LAYER_BSTRIDE[2],
                                    _LAYER_BSTRIDE[3])))
            flats.append(off + b * bst + ps * _NB + bx)
            labs.append(lb)

        for v in range(4):
            dup = jnp.zeros((16,), jnp.bool_)
            for k in range(1, 16):
                rolled = _take16(flats[v], (iota + k) & 15)
                dup = dup | ((rolled == flats[v]) & (iota < 16 - k))
            for u in range(v + 1, 4):
                for k in range(16):
                    rolled = _take16(flats[u], (iota + k) & 15)
                    dup = dup | (rolled == flats[v])
            win = jnp.logical_not(dup)
            f = flats[v]
            sl = pl.ds(16 * v, 16)
            s_g[sl] = f >> 3
            s_sub[sl] = f & 7
            s_lab[sl] = labs[v]
            s_win[sl] = win.astype(jnp.float32)
            s_pos[sl] = (win & (labs[v] > 0)).astype(jnp.float32)
            for c, stc in enumerate((s_t0, s_t1, s_t2, s_t3)):
                gtc = gtv[c, sl]
                dfc = dfv[c, sl]
                stc[sl] = (gtc - dfc) / jnp.float32(0.1)

        base = w * _NOBJ
        outs = (o_g, o_sub, o_lab, o_win, o_pos, o_t0, o_t1, o_t2, o_t3)
        scr = (s_g, s_sub, s_lab, s_win, s_pos, s_t0, s_t1, s_t2, s_t3)
        for o, s in zip(outs, scr):
            pltpu.sync_copy(s, o.at[pl.ds(base, _NOBJ)])

        @pl.when(w == 0)
        def _():
            for t in range(8):
                zi[pl.ds(16 * t, 16)] = jnp.zeros((16,), jnp.int32)
                zf[pl.ds(16 * t, 16)] = jnp.zeros((16,), jnp.float32)
            for o in (o_g, o_sub, o_lab):
                pltpu.sync_copy(zi, o.at[pl.ds(_B * _NOBJ, 128)])
            for o in (o_win, o_pos, o_t0, o_t1, o_t2, o_t3):
                pltpu.sync_copy(zf, o.at[pl.ds(_B * _NOBJ, 128)])


def kernel(Loc, Cls, Seg, gt_box_batch, df_box_batch, idx_batch, cls_batch,
           bat_s, mining, seg_label):
    idxt = jnp.transpose(idx_batch[..., 1:].astype(jnp.int32), (2, 0, 1))
    gtt = jnp.transpose(gt_box_batch, (2, 0, 1))
    dft = jnp.transpose(df_box_batch, (2, 0, 1))
    mesh = plsc.VectorSubcoreMesh(core_axis_name="c", subcore_axis_name="s")
    i32v = jax.ShapeDtypeStruct((_NPAD,), jnp.int32)
    f32v = jax.ShapeDtypeStruct((_NPAD,), jnp.float32)
    gidx, sub, lab, win, pos, t0, t1, t2, t3 = pl.kernel(
        _sc_body,
        mesh=mesh,
        compiler_params=pltpu.CompilerParams(needs_layout_passes=False),
        out_type=(i32v, i32v, i32v, f32v, f32v, f32v, f32v, f32v, f32v),
        scratch_types=(
            [pltpu.VMEM((_NOBJ,), jnp.int32)] * 4
            + [pltpu.VMEM((4, _NOBJ), jnp.float32)] * 2
            + [pltpu.VMEM((_NOBJ,), jnp.int32)] * 3
            + [pltpu.VMEM((_NOBJ,), jnp.float32)] * 6
            + [pltpu.VMEM((128,), jnp.int32),
               pltpu.VMEM((128,), jnp.float32)]
        ),
    )(idxt, cls_batch.astype(jnp.int32), gtt, dft)

    def _seg_i(i):
        j = jnp.minimum(i, _SEG_STEPS - 1)
        return j // (_SEG_H // _SEG_BH), j % (_SEG_H // _SEG_BH)

    def _seg_map(i, g_ref):
        bi, hi = _seg_i(i)
        return (bi, 0, hi, 0)

    def _lab_map(i, g_ref):
        bi, hi = _seg_i(i)
        return (bi, hi, 0)

    acc = pl.pallas_call(
        _fused_body,
        grid_spec=pltpu.PrefetchScalarGridSpec(
            num_scalar_prefetch=1,
            grid=(_STEPS,),
            in_specs=[
                pl.BlockSpec(
                    (_CLS_RB, _NCLS),
                    functools.partial(
                        lambda i, g_ref, s: (jnp.minimum(i, _CLS_STEPS - 1)
                                             + s * _CLS_STEPS, 0), s=s))
                for s in range(_NSPLIT)
            ] + [
                pl.BlockSpec((1, _NCLS, _SEG_BH, _SEG_H), _seg_map),
                pl.BlockSpec((1, _SEG_BH, _SEG_H), _lab_map),
            ] + [
                pl.BlockSpec(
                    (8, _NCLS),
                    functools.partial(
                        lambda i, g_ref, j: (g_ref[i * _K + j], 0), j=j))
                for j in range(_K)
            ] + [
                pl.BlockSpec(
                    (8, 4),
                    functools.partial(
                        lambda i, g_ref, j: (g_ref[i * _K + j], 0), j=j))
                for j in range(_K)
            ] + [pl.BlockSpec(memory_space=pltpu.SMEM)] * 8,
            out_specs=pl.BlockSpec((1, 8), lambda i, g_ref: (0, 0),
                                   memory_space=pltpu.SMEM),
        ),
        out_shape=jax.ShapeDtypeStruct((1, 8), jnp.float32),
    )(gidx, *([Cls] * _NSPLIT), Seg, seg_label.astype(jnp.int32),
      *([Cls] * _K), *([Loc] * _K),
      sub, lab, win, pos, t0, t1, t2, t3)

    cls_loss = (acc[0, 0] + acc[0, 2]) / jnp.float32(_TOTAL)
    loc_loss = acc[0, 3] / jnp.maximum(acc[0, 4], 1.0)
    seg_loss = acc[0, 1] / jnp.float32(_B * _SEG_H * _SEG_H)
    return cls_loss + loc_loss + seg_loss


# speedup vs baseline: 2.6339x; 1.0105x over previous
import functools

import jax
import jax.numpy as jnp
import numpy as np
from jax import lax
from jax.experimental import pallas as pl
from jax.experimental.pallas import tpu as pltpu
from jax.experimental.pallas import tpu_sc as plsc

_MAP_SIZES = [64, 32, 16, 8, 4, 2]
_NB = 6
_B = 16
_NOBJ = 64
_NCLS = 21
_SEG_H = 256
_TOTAL = sum(_B * _NB * ms * ms for ms in _MAP_SIZES)
_NSPLIT = 3
_CLS_STEPS = 91
_CLS_RB = _TOTAL // (_NSPLIT * _CLS_STEPS)
_SEG_BH = 64
_SEG_STEPS = _B * (_SEG_H // _SEG_BH)
_K = 12
_STEPS = 96
_NPAD = _STEPS * _K

_LAYER_OFF = [0, 393216, 491520, 516096]
_LAYER_BSTRIDE = [24576, 6144, 1536, 384]


def _dense_body(*refs):
    xs = refs[:_NSPLIT]
    seg_ref, lab_ref = refs[_NSPLIT:_NSPLIT + 2]
    acc_ref = refs[-1]
    i = pl.program_id(0)

    @pl.when(i == 0)
    def _():
        acc_ref[0, 0] = 0.0
        acc_ref[0, 1] = 0.0

    total = jnp.float32(0.0)
    for x_ref in xs:
        x = x_ref[...]
        s = jnp.sum(jnp.exp(x), axis=1)
        total = total + jnp.sum(jnp.log(s)) - jnp.sum(x[:, 0])
    acc_ref[0, 0] += total

    @pl.when(i < _SEG_STEPS)
    def _():
        lab = lab_ref[0]
        x0 = seg_ref[0, 0]
        se = jnp.exp(x0)
        xl = jnp.where(lab == 0, x0, 0.0)
        for c in range(1, _NCLS):
            xc = seg_ref[0, c]
            se = se + jnp.exp(xc)
            xl = jnp.where(lab == c, xc, xl)
        acc_ref[0, 1] += jnp.sum(jnp.log(se)) - jnp.sum(xl)


_CORR_K = 32


def _corr_body(g_ref, *refs):
    cbs = refs[:_CORR_K]
    lbs = refs[_CORR_K:2 * _CORR_K]
    (sub_r, lab_r, win_r, pos_r, t0_r, t1_r, t2_r, t3_r) = \
        refs[2 * _CORR_K:-1]
    acc_ref = refs[-1]
    i = pl.program_id(0)

    @pl.when(i == 0)
    def _():
        for c in range(8):
            acc_ref[0, c] = 0.0

    lane = lax.broadcasted_iota(jnp.int32, (1, _NCLS), 1)
    lane4 = lax.broadcasted_iota(jnp.int32, (1, 4), 1)
    a_cls = jnp.float32(0.0)
    a_loc = jnp.float32(0.0)
    a_n = jnp.float32(0.0)
    for j in range(_CORR_K):
        idx = i * _CORR_K + j
        sub = sub_r[idx]
        lab = lab_r[idx]
        w = win_r[idx]
        p = pos_r[idx]
        x = cbs[j][pl.ds(sub, 1), :]
        x0 = jnp.sum(jnp.where(lane == 0, x, 0.0))
        xl = jnp.sum(jnp.where(lane == lab, x, 0.0))
        a_cls = a_cls + w * (x0 - xl)
        l = lbs[j][pl.ds(sub, 1), :]
        t = jnp.where(lane4 == 0, t0_r[idx],
                      jnp.where(lane4 == 1, t1_r[idx],
                                jnp.where(lane4 == 2, t2_r[idx], t3_r[idx])))
        d = jnp.abs(l - t)
        a_loc = a_loc + p * jnp.sum(jnp.where(d < 1.0, 0.5 * d * d, d - 0.5))
        a_n = a_n + p
    acc_ref[0, 2] += a_cls
    acc_ref[0, 3] += a_loc
    acc_ref[0, 4] += a_n


def _take16(x, idx):
    dnums = lax.GatherDimensionNumbers(
        offset_dims=(), collapsed_slice_dims=(0,), start_index_map=(0,))
    return lax.gather(x, idx[:, None], dnums, slice_sizes=(1,),
                      mode=lax.GatherScatterMode.PROMISE_IN_BOUNDS)


def _sc_body(idxt, clsb, gtt, dft,
             o_g, o_sub, o_lab, o_win, o_pos, o_t0, o_t1, o_t2, o_t3,
             liv, piv, biv, cbv, gtv, dfv,
             s_g, s_sub, s_lab, s_win, s_pos, s_t0, s_t1, s_t2, s_t3,
             zi, zf):
    w = lax.axis_index("s") * 2 + lax.axis_index("c")

    @pl.when(w < _B)
    def _():
        b = w
        pltpu.sync_copy(idxt.at[0, b], liv)
        pltpu.sync_copy(idxt.at[1, b], piv)
        pltpu.sync_copy(idxt.at[2, b], biv)
        pltpu.sync_copy(clsb.at[b], cbv)
        for c in range(4):
            pltpu.sync_copy(gtt.at[c, b], gtv.at[c])
            pltpu.sync_copy(dft.at[c, b], dfv.at[c])

        iota = lax.iota(jnp.int32, 16)
        flats = []
        labs = []
        for v in range(4):
            ly = liv[pl.ds(16 * v, 16)]
            ps = piv[pl.ds(16 * v, 16)]
            bx = biv[pl.ds(16 * v, 16)]
            lb = cbv[pl.ds(16 * v, 16)]
            off = jnp.where(
                ly == 0, _LAYER_OFF[0],
                jnp.where(ly == 1, _LAYER_OFF[1],
                          jnp.where(ly == 2, _LAYER_OFF[2], _LAYER_OFF[3])))
            bst = jnp.where(
                ly == 0, _LAYER_BSTRIDE[0],
                jnp.where(ly == 1, _LAYER_BSTRIDE[1],
                          jnp.where(ly == 2, _LAYER_BSTRIDE[2],
                                    _LAYER_BSTRIDE[3])))
            flats.append(off + b * bst + ps * _NB + bx)
            labs.append(lb)

        for v in range(4):
            dup = jnp.zeros((16,), jnp.bool_)
            for k in range(1, 16):
                rolled = _take16(flats[v], (iota + k) & 15)
                dup = dup | ((rolled == flats[v]) & (iota < 16 - k))
            for u in range(v + 1, 4):
                for k in range(16):
                    rolled = _take16(flats[u], (iota + k) & 15)
                    dup = dup | (rolled == flats[v])
            win = jnp.logical_not(dup)
            f = flats[v]
            sl = pl.ds(16 * v, 16)
            s_g[sl] = f >> 3
            s_sub[sl] = f & 7
            s_lab[sl] = labs[v]
            s_win[sl] = win.astype(jnp.float32)
            s_pos[sl] = (win & (labs[v] > 0)).astype(jnp.float32)
            for c, stc in enumerate((s_t0, s_t1, s_t2, s_t3)):
                gtc = gtv[c, sl]
                dfc = dfv[c, sl]
                stc[sl] = (gtc - dfc) / jnp.float32(0.1)

        base = w * _NOBJ
        outs = (o_g, o_sub, o_lab, o_win, o_pos, o_t0, o_t1, o_t2, o_t3)
        scr = (s_g, s_sub, s_lab, s_win, s_pos, s_t0, s_t1, s_t2, s_t3)
        for o, s in zip(outs, scr):
            pltpu.sync_copy(s, o.at[pl.ds(base, _NOBJ)])

        @pl.when(w == 0)
        def _():
            for t in range(8):
                zi[pl.ds(16 * t, 16)] = jnp.zeros((16,), jnp.int32)
                zf[pl.ds(16 * t, 16)] = jnp.zeros((16,), jnp.float32)
            for o in (o_g, o_sub, o_lab):
                pltpu.sync_copy(zi, o.at[pl.ds(_B * _NOBJ, 128)])
            for o in (o_win, o_pos, o_t0, o_t1, o_t2, o_t3):
                pltpu.sync_copy(zf, o.at[pl.ds(_B * _NOBJ, 128)])


def kernel(Loc, Cls, Seg, gt_box_batch, df_box_batch, idx_batch, cls_batch,
           bat_s, mining, seg_label):
    idxt = jnp.transpose(idx_batch[..., 1:].astype(jnp.int32), (2, 0, 1))
    gtt = jnp.transpose(gt_box_batch, (2, 0, 1))
    dft = jnp.transpose(df_box_batch, (2, 0, 1))
    mesh = plsc.VectorSubcoreMesh(core_axis_name="c", subcore_axis_name="s")
    i32v = jax.ShapeDtypeStruct((_NPAD,), jnp.int32)
    f32v = jax.ShapeDtypeStruct((_NPAD,), jnp.float32)
    gidx, sub, lab, win, pos, t0, t1, t2, t3 = pl.kernel(
        _sc_body,
        mesh=mesh,
        compiler_params=pltpu.CompilerParams(needs_layout_passes=False),
        out_type=(i32v, i32v, i32v, f32v, f32v, f32v, f32v, f32v, f32v),
        scratch_types=(
            [pltpu.VMEM((_NOBJ,), jnp.int32)] * 4
            + [pltpu.VMEM((4, _NOBJ), jnp.float32)] * 2
            + [pltpu.VMEM((_NOBJ,), jnp.int32)] * 3
            + [pltpu.VMEM((_NOBJ,), jnp.float32)] * 6
            + [pltpu.VMEM((128,), jnp.int32),
               pltpu.VMEM((128,), jnp.float32)]
        ),
    )(idxt, cls_batch.astype(jnp.int32), gtt, dft)

    def _seg_i(i):
        j = jnp.minimum(i, _SEG_STEPS - 1)
        return j // (_SEG_H // _SEG_BH), j % (_SEG_H // _SEG_BH)

    def _seg_map(i):
        bi, hi = _seg_i(i)
        return (bi, 0, hi, 0)

    def _lab_map(i):
        bi, hi = _seg_i(i)
        return (bi, hi, 0)

    dacc = pl.pallas_call(
        _dense_body,
        grid=(_CLS_STEPS,),
        in_specs=[
            pl.BlockSpec(
                (_CLS_RB, _NCLS),
                functools.partial(lambda i, s: (i + s * _CLS_STEPS, 0), s=s))
            for s in range(_NSPLIT)
        ] + [
            pl.BlockSpec((1, _NCLS, _SEG_BH, _SEG_H), _seg_map),
            pl.BlockSpec((1, _SEG_BH, _SEG_H), _lab_map),
        ],
        out_specs=pl.BlockSpec((1, 2), lambda i: (0, 0),
                               memory_space=pltpu.SMEM),
        out_shape=jax.ShapeDtypeStruct((1, 2), jnp.float32),
    )(*([Cls] * _NSPLIT), Seg, seg_label.astype(jnp.int32))

    acc = pl.pallas_call(
        _corr_body,
        grid_spec=pltpu.PrefetchScalarGridSpec(
            num_scalar_prefetch=1,
            grid=(_NPAD // _CORR_K,),
            in_specs=[
                pl.BlockSpec(
                    (8, _NCLS),
                    functools.partial(
                        lambda i, g_ref, j: (g_ref[i * _CORR_K + j], 0), j=j))
                for j in range(_CORR_K)
            ] + [
                pl.BlockSpec(
                    (8, 4),
                    functools.partial(
                        lambda i, g_ref, j: (g_ref[i * _CORR_K + j], 0), j=j))
                for j in range(_CORR_K)
            ] + [pl.BlockSpec(memory_space=pltpu.SMEM)] * 8,
            out_specs=pl.BlockSpec((1, 8), lambda i, g_ref: (0, 0),
                                   memory_space=pltpu.SMEM),
        ),
        out_shape=jax.ShapeDtypeStruct((1, 8), jnp.float32),
    )(gidx, *([Cls] * _CORR_K), *([Loc] * _CORR_K),
      sub, lab, win, pos, t0, t1, t2, t3)

    cls_loss = (dacc[0, 0] + acc[0, 2]) / jnp.float32(_TOTAL)
    loc_loss = acc[0, 3] / jnp.maximum(acc[0, 4], 1.0)
    seg_loss = dacc[0, 1] / jnp.float32(_B * _SEG_H * _SEG_H)
    return cls_loss + loc_loss + seg_loss


# speedup vs baseline: 2.8342x; 1.0760x over previous
import functools

import jax
import jax.numpy as jnp
import numpy as np
from jax import lax
from jax.experimental import pallas as pl
from jax.experimental.pallas import tpu as pltpu
from jax.experimental.pallas import tpu_sc as plsc

_MAP_SIZES = [64, 32, 16, 8, 4, 2]
_NB = 6
_B = 16
_NOBJ = 64
_NCLS = 21
_SEG_H = 256
_TOTAL = sum(_B * _NB * ms * ms for ms in _MAP_SIZES)
_NSPLIT = 1
_CLS_STEPS = 91
_CLS_RB = _TOTAL // (_NSPLIT * _CLS_STEPS)
_SEG_BH = 64
_SEG_STEPS = _B * (_SEG_H // _SEG_BH)
_K = 12
_STEPS = 96
_NPAD = _STEPS * _K

_LAYER_OFF = [0, 393216, 491520, 516096]
_LAYER_BSTRIDE = [24576, 6144, 1536, 384]


def _dense_body(*refs):
    xs = refs[:_NSPLIT]
    seg_ref, lab_ref = refs[_NSPLIT:_NSPLIT + 2]
    acc_ref = refs[-1]
    i = pl.program_id(0)

    @pl.when(i == 0)
    def _():
        acc_ref[0, 0] = 0.0
        acc_ref[0, 1] = 0.0

    total = jnp.float32(0.0)
    for x_ref in xs:
        x = x_ref[...]
        s = jnp.sum(jnp.exp(x), axis=1)
        total = total + jnp.sum(jnp.log(s)) - jnp.sum(x[:, 0])
    acc_ref[0, 0] += total

    @pl.when(i < _SEG_STEPS)
    def _():
        lab = lab_ref[0]
        x0 = seg_ref[0, 0]
        se = jnp.exp(x0)
        xl = jnp.where(lab == 0, x0, 0.0)
        for c in range(1, _NCLS):
            xc = seg_ref[0, c]
            se = se + jnp.exp(xc)
            xl = jnp.where(lab == c, xc, xl)
        acc_ref[0, 1] += jnp.sum(jnp.log(se)) - jnp.sum(xl)


_CORR_K = 32


def _corr_body(g_ref, *refs):
    cbs = refs[:_CORR_K]
    lbs = refs[_CORR_K:2 * _CORR_K]
    (sub_r, lab_r, win_r, pos_r, t0_r, t1_r, t2_r, t3_r) = \
        refs[2 * _CORR_K:-1]
    acc_ref = refs[-1]
    i = pl.program_id(0)

    @pl.when(i == 0)
    def _():
        for c in range(8):
            acc_ref[0, c] = 0.0

    lane = lax.broadcasted_iota(jnp.int32, (1, _NCLS), 1)
    lane4 = lax.broadcasted_iota(jnp.int32, (1, 4), 1)
    a_cls = jnp.float32(0.0)
    a_loc = jnp.float32(0.0)
    a_n = jnp.float32(0.0)
    for j in range(_CORR_K):
        idx = i * _CORR_K + j
        sub = sub_r[idx]
        lab = lab_r[idx]
        w = win_r[idx]
        p = pos_r[idx]
        x = cbs[j][pl.ds(sub, 1), :]
        x0 = jnp.sum(jnp.where(lane == 0, x, 0.0))
        xl = jnp.sum(jnp.where(lane == lab, x, 0.0))
        a_cls = a_cls + w * (x0 - xl)
        l = lbs[j][pl.ds(sub, 1), :]
        t = jnp.where(lane4 == 0, t0_r[idx],
                      jnp.where(lane4 == 1, t1_r[idx],
                                jnp.where(lane4 == 2, t2_r[idx], t3_r[idx])))
        d = jnp.abs(l - t)
        a_loc = a_loc + p * jnp.sum(jnp.where(d < 1.0, 0.5 * d * d, d - 0.5))
        a_n = a_n + p
    acc_ref[0, 2] += a_cls
    acc_ref[0, 3] += a_loc
    acc_ref[0, 4] += a_n


def _take16(x, idx):
    dnums = lax.GatherDimensionNumbers(
        offset_dims=(), collapsed_slice_dims=(0,), start_index_map=(0,))
    return lax.gather(x, idx[:, None], dnums, slice_sizes=(1,),
                      mode=lax.GatherScatterMode.PROMISE_IN_BOUNDS)


def _sc_body(idxt, clsb, gtt, dft,
             o_g, o_sub, o_lab, o_win, o_pos, o_t0, o_t1, o_t2, o_t3,
             liv, piv, biv, cbv, gtv, dfv,
             s_g, s_sub, s_lab, s_win, s_pos, s_t0, s_t1, s_t2, s_t3,
             zi, zf):
    w = lax.axis_index("s") * 2 + lax.axis_index("c")

    @pl.when(w < _B)
    def _():
        b = w
        pltpu.sync_copy(idxt.at[0, b], liv)
        pltpu.sync_copy(idxt.at[1, b], piv)
        pltpu.sync_copy(idxt.at[2, b], biv)
        pltpu.sync_copy(clsb.at[b], cbv)
        for c in range(4):
            pltpu.sync_copy(gtt.at[c, b], gtv.at[c])
            pltpu.sync_copy(dft.at[c, b], dfv.at[c])

        iota = lax.iota(jnp.int32, 16)
        flats = []
        labs = []
        for v in range(4):
            ly = liv[pl.ds(16 * v, 16)]
            ps = piv[pl.ds(16 * v, 16)]
            bx = biv[pl.ds(16 * v, 16)]
            lb = cbv[pl.ds(16 * v, 16)]
            off = jnp.where(
                ly == 0, _LAYER_OFF[0],
                jnp.where(ly == 1, _LAYER_OFF[1],
                          jnp.where(ly == 2, _LAYER_OFF[2], _LAYER_OFF[3])))
            bst = jnp.where(
                ly == 0, _LAYER_BSTRIDE[0],
                jnp.where(ly == 1, _LAYER_BSTRIDE[1],
                          jnp.where(ly == 2, _LAYER_BSTRIDE[2],
                                    _LAYER_BSTRIDE[3])))
            flats.append(off + b * bst + ps * _NB + bx)
            labs.append(lb)

        for v in range(4):
            dup = jnp.zeros((16,), jnp.bool_)
            for k in range(1, 16):
                rolled = _take16(flats[v], (iota + k) & 15)
                dup = dup | ((rolled == flats[v]) & (iota < 16 - k))
            for u in range(v + 1, 4):
                for k in range(16):
                    rolled = _take16(flats[u], (iota + k) & 15)
                    dup = dup | (rolled == flats[v])
            win = jnp.logical_not(dup)
            f = flats[v]
            sl = pl.ds(16 * v, 16)
            s_g[sl] = f >> 3
            s_sub[sl] = f & 7
            s_lab[sl] = labs[v]
            s_win[sl] = win.astype(jnp.float32)
            s_pos[sl] = (win & (labs[v] > 0)).astype(jnp.float32)
            for c, stc in enumerate((s_t0, s_t1, s_t2, s_t3)):
                gtc = gtv[c, sl]
                dfc = dfv[c, sl]
                stc[sl] = (gtc - dfc) / jnp.float32(0.1)

        base = w * _NOBJ
        outs = (o_g, o_sub, o_lab, o_win, o_pos, o_t0, o_t1, o_t2, o_t3)
        scr = (s_g, s_sub, s_lab, s_win, s_pos, s_t0, s_t1, s_t2, s_t3)
        for o, s in zip(outs, scr):
            pltpu.sync_copy(s, o.at[pl.ds(base, _NOBJ)])

        @pl.when(w == 0)
        def _():
            for t in range(8):
                zi[pl.ds(16 * t, 16)] = jnp.zeros((16,), jnp.int32)
                zf[pl.ds(16 * t, 16)] = jnp.zeros((16,), jnp.float32)
            for o in (o_g, o_sub, o_lab):
                pltpu.sync_copy(zi, o.at[pl.ds(_B * _NOBJ, 128)])
            for o in (o_win, o_pos, o_t0, o_t1, o_t2, o_t3):
                pltpu.sync_copy(zf, o.at[pl.ds(_B * _NOBJ, 128)])


def kernel(Loc, Cls, Seg, gt_box_batch, df_box_batch, idx_batch, cls_batch,
           bat_s, mining, seg_label):
    idxt = jnp.transpose(idx_batch[..., 1:].astype(jnp.int32), (2, 0, 1))
    gtt = jnp.transpose(gt_box_batch, (2, 0, 1))
    dft = jnp.transpose(df_box_batch, (2, 0, 1))
    mesh = plsc.VectorSubcoreMesh(core_axis_name="c", subcore_axis_name="s")
    i32v = jax.ShapeDtypeStruct((_NPAD,), jnp.int32)
    f32v = jax.ShapeDtypeStruct((_NPAD,), jnp.float32)
    gidx, sub, lab, win, pos, t0, t1, t2, t3 = pl.kernel(
        _sc_body,
        mesh=mesh,
        compiler_params=pltpu.CompilerParams(needs_layout_passes=False),
        out_type=(i32v, i32v, i32v, f32v, f32v, f32v, f32v, f32v, f32v),
        scratch_types=(
            [pltpu.VMEM((_NOBJ,), jnp.int32)] * 4
            + [pltpu.VMEM((4, _NOBJ), jnp.float32)] * 2
            + [pltpu.VMEM((_NOBJ,), jnp.int32)] * 3
            + [pltpu.VMEM((_NOBJ,), jnp.float32)] * 6
            + [pltpu.VMEM((128,), jnp.int32),
               pltpu.VMEM((128,), jnp.float32)]
        ),
    )(idxt, cls_batch.astype(jnp.int32), gtt, dft)

    def _seg_i(i):
        j = jnp.minimum(i, _SEG_STEPS - 1)
        return j // (_SEG_H // _SEG_BH), j % (_SEG_H // _SEG_BH)

    def _seg_map(i):
        bi, hi = _seg_i(i)
        return (bi, 0, hi, 0)

    def _lab_map(i):
        bi, hi = _seg_i(i)
        return (bi, hi, 0)

    dacc = pl.pallas_call(
        _dense_body,
        grid=(_CLS_STEPS,),
        in_specs=[
            pl.BlockSpec(
                (_CLS_RB, _NCLS),
                functools.partial(lambda i, s: (i + s * _CLS_STEPS, 0), s=s))
            for s in range(_NSPLIT)
        ] + [
            pl.BlockSpec((1, _NCLS, _SEG_BH, _SEG_H), _seg_map),
            pl.BlockSpec((1, _SEG_BH, _SEG_H), _lab_map),
        ],
        out_specs=pl.BlockSpec((1, 2), lambda i: (0, 0),
                               memory_space=pltpu.SMEM),
        out_shape=jax.ShapeDtypeStruct((1, 2), jnp.float32),
    )(*([Cls] * _NSPLIT), Seg, seg_label.astype(jnp.int32))

    acc = pl.pallas_call(
        _corr_body,
        grid_spec=pltpu.PrefetchScalarGridSpec(
            num_scalar_prefetch=1,
            grid=(_NPAD // _CORR_K,),
            in_specs=[
                pl.BlockSpec(
                    (8, _NCLS),
                    functools.partial(
                        lambda i, g_ref, j: (g_ref[i * _CORR_K + j], 0), j=j))
                for j in range(_CORR_K)
            ] + [
                pl.BlockSpec(
                    (8, 4),
                    functools.partial(
                        lambda i, g_ref, j: (g_ref[i * _CORR_K + j], 0), j=j))
                for j in range(_CORR_K)
            ] + [pl.BlockSpec(memory_space=pltpu.SMEM)] * 8,
            out_specs=pl.BlockSpec((1, 8), lambda i, g_ref: (0, 0),
                                   memory_space=pltpu.SMEM),
        ),
        out_shape=jax.ShapeDtypeStruct((1, 8), jnp.float32),
    )(gidx, *([Cls] * _CORR_K), *([Loc] * _CORR_K),
      sub, lab, win, pos, t0, t1, t2, t3)

    cls_loss = (dacc[0, 0] + acc[0, 2]) / jnp.float32(_TOTAL)
    loc_loss = acc[0, 3] / jnp.maximum(acc[0, 4], 1.0)
    seg_loss = dacc[0, 1] / jnp.float32(_B * _SEG_H * _SEG_H)
    return cls_loss + loc_loss + seg_loss
